# baseline (speedup 1.0000x reference)
import functools

import jax
import jax.numpy as jnp
import numpy as np
from jax import lax
from jax.experimental import pallas as pl
from jax.experimental.pallas import tpu as pltpu
from jax.experimental.pallas import tpu_sc as plsc

_K = 8
_TC_ROWS = 96
_SC_ROWS = 32
_LANES = 16


_TC_CHUNK = 2048


def _tc_body(s_ref, g_ref, o_ref):
    rows, cols = s_ref.shape
    n_ch = cols // _TC_CHUNK
    sls = [slice(c * _TC_CHUNK, (c + 1) * _TC_CHUNK) for c in range(n_ch)]
    u = []
    parts = []
    for sl in sls:
        s = s_ref[:, sl] + g_ref[:, sl]
        u.append(s)
        parts.append(jnp.max(s, axis=1, keepdims=True))
    m = functools.reduce(jnp.maximum, parts)
    parts = []
    for c, sl in enumerate(sls):
        e = jnp.exp(u[c] - m)
        u[c] = e
        parts.append(jnp.sum(e, axis=1, keepdims=True))
    denom = functools.reduce(jnp.add, parts)
    khot = [None] * n_ch
    for t in range(_K):
        r = 1.0 / denom
        parts = []
        for c in range(n_ch):
            onehot = u[c] * r
            khot[c] = onehot if khot[c] is None else khot[c] + onehot
            if t < _K - 1:
                un = u[c] - u[c] * onehot
                u[c] = un
                parts.append(jnp.sum(un, axis=1, keepdims=True))
        if t < _K - 1:
            denom = functools.reduce(jnp.add, parts)
    for c, sl in enumerate(sls):
        o_ref[:, sl] = khot[c]


def _tc_call(scores, g):
    n_rows, n_cols = scores.shape
    block_rows = 32
    spec = pl.BlockSpec((block_rows, n_cols), lambda i: (i, 0))
    return pl.pallas_call(
        _tc_body,
        grid=(_TC_ROWS // block_rows,),
        in_specs=[spec, spec],
        out_specs=spec,
        out_shape=jax.ShapeDtypeStruct((n_rows, n_cols), jnp.float32),
    )(scores, g)


_CHUNK = 8
_NACC = 4


def _zeros4():
    z = jnp.zeros((_LANES,), jnp.float32)
    return (z, z, z, z)


def _merge4(zs):
    return (zs[0] + zs[1]) + (zs[2] + zs[3])


def _sc_worker_body(n_cols, row_base, scores_hbm, g_hbm, out_hbm,
                    a_v, b_v, sem_a, sem_b):
    wid = lax.axis_index("s") * 2 + lax.axis_index("c")
    row = row_base + wid
    n_trip = n_cols // (_LANES * _CHUNK)

    cp_a = pltpu.async_copy(scores_hbm.at[row], a_v, sem_a)
    cp_b = pltpu.async_copy(g_hbm.at[row], b_v, sem_b)
    cp_a.wait()
    cp_b.wait()

    @plsc.parallel_loop(0, n_trip, 1, carry=_zeros4())
    def mvecs(i, ms):
        base = i * (_CHUNK * _LANES)
        ms = list(ms)
        for j in range(_CHUNK):
            sl = pl.ds(base + j * _LANES, _LANES)
            x = a_v[sl] + b_v[sl]
            a_v[sl] = x
            ms[j % _NACC] = jnp.maximum(ms[j % _NACC], x)
        return tuple(ms)

    m = jnp.max(jnp.maximum(jnp.maximum(mvecs[0], mvecs[1]),
                            jnp.maximum(mvecs[2], mvecs[3])))

    @plsc.parallel_loop(0, n_trip, 1, carry=_zeros4())
    def zvecs(i, zs):
        base = i * (_CHUNK * _LANES)
        zs = list(zs)
        for j in range(_CHUNK):
            sl = pl.ds(base + j * _LANES, _LANES)
            e = jnp.exp(a_v[sl] - m)
            a_v[sl] = e
            zs[j % _NACC] = zs[j % _NACC] + e
        return tuple(zs)

    z = jnp.sum(_merge4(zvecs))

    c = 1.0 / jnp.full((_LANES,), z, jnp.float32)

    @plsc.parallel_loop(0, n_trip, 1, carry=_zeros4())
    def zvecs(i, zs, c=c):
        base = i * (_CHUNK * _LANES)
        zs = list(zs)
        for j in range(_CHUNK):
            sl = pl.ds(base + j * _LANES, _LANES)
            u = a_v[sl]
            v = u * c
            b_v[sl] = v
            un = u - u * v
            a_v[sl] = un
            zs[j % _NACC] = zs[j % _NACC] + un
        return tuple(zs)

    z = jnp.sum(_merge4(zvecs))

    def mid_round(_, z):
        c = 1.0 / jnp.full((_LANES,), z, jnp.float32)

        @plsc.parallel_loop(0, n_trip, 1, carry=_zeros4())
        def zvecs(i, zs, c=c):
            base = i * (_CHUNK * _LANES)
            zs = list(zs)
            for j in range(_CHUNK):
                sl = pl.ds(base + j * _LANES, _LANES)
                u = a_v[sl]
                v = u * c
                b_v[sl] = b_v[sl] + v
                un = u - u * v
                a_v[sl] = un
                zs[j % _NACC] = zs[j % _NACC] + un
            return tuple(zs)

        return jnp.sum(_merge4(zvecs))

    z = lax.fori_loop(1, _K - 1, mid_round, z)

    c = 1.0 / jnp.full((_LANES,), z, jnp.float32)

    @plsc.parallel_loop(0, n_trip, 1)
    def _(i, c=c):
        base = i * (_CHUNK * _LANES)
        for j in range(_CHUNK):
            sl = pl.ds(base + j * _LANES, _LANES)
            b_v[sl] = b_v[sl] + a_v[sl] * c

    pltpu.sync_copy(b_v, out_hbm.at[wid])


def _sc_call(scores, g):
    n_rows, n_cols = scores.shape
    mesh = plsc.VectorSubcoreMesh(core_axis_name="c", subcore_axis_name="s")
    kfn = functools.partial(
        pl.kernel,
        mesh=mesh,
        compiler_params=pltpu.CompilerParams(needs_layout_passes=False),
        out_type=jax.ShapeDtypeStruct((_SC_ROWS, n_cols), jnp.float32),
        scratch_types=[
            pltpu.VMEM((n_cols,), jnp.float32),
            pltpu.VMEM((n_cols,), jnp.float32),
            pltpu.SemaphoreType.DMA,
            pltpu.SemaphoreType.DMA,
        ],
    )(functools.partial(_sc_worker_body, n_cols, n_rows - _SC_ROWS))
    return kfn(scores, g)


@jax.jit
def kernel(scores, g):
    if _SC_ROWS == 0:
        return _tc_call(scores, g)
    sc_out = _sc_call(scores, g)
    tc_out = _tc_call(scores, g)
    return lax.dynamic_update_slice(tc_out, sc_out, (_TC_ROWS, 0))


# speedup vs baseline: 1.4378x; 1.4378x over previous
import functools

import jax
import jax.numpy as jnp
from jax.experimental import pallas as pl

_K = 8
_CHUNK = 2048
_BLOCK_ROWS = 32


def _body(s_ref, g_ref, o_ref):
    rows, cols = s_ref.shape
    n_ch = cols // _CHUNK
    sls = [slice(c * _CHUNK, (c + 1) * _CHUNK) for c in range(n_ch)]
    u = []
    parts = []
    for sl in sls:
        s = s_ref[:, sl] + g_ref[:, sl]
        u.append(s)
        parts.append(jnp.max(s, axis=1, keepdims=True))
    m = functools.reduce(jnp.maximum, parts)
    parts = []
    for c in range(n_ch):
        e = jnp.exp(u[c] - m)
        u[c] = e
        parts.append(jnp.sum(e, axis=1, keepdims=True))
    denom = functools.reduce(jnp.add, parts)
    khot = [None] * n_ch
    for t in range(_K):
        r = 1.0 / denom
        parts = []
        for c in range(n_ch):
            onehot = u[c] * r
            khot[c] = onehot if khot[c] is None else khot[c] + onehot
            if t < _K - 1:
                un = u[c] - u[c] * onehot
                u[c] = un
                parts.append(jnp.sum(un, axis=1, keepdims=True))
        if t < _K - 1:
            denom = functools.reduce(jnp.add, parts)
    for c, sl in enumerate(sls):
        o_ref[:, sl] = khot[c]


@jax.jit
def kernel(scores, g):
    n_rows, n_cols = scores.shape
    spec = pl.BlockSpec((_BLOCK_ROWS, n_cols), lambda i: (i, 0))
    return pl.pallas_call(
        _body,
        grid=(n_rows // _BLOCK_ROWS,),
        in_specs=[spec, spec],
        out_specs=spec,
        out_shape=jax.ShapeDtypeStruct((n_rows, n_cols), jnp.float32),
    )(scores, g)
